# baseline (speedup 1.0000x reference)
import functools

import jax
import jax.numpy as jnp
from jax.experimental import pallas as pl
from jax.experimental.pallas import tpu as pltpu

_NUM_PARTS = 6


def _tgam_kernel(x_ref, w_ref, b_ref, o_ref):
    L = x_ref.shape[1]
    C = x_ref.shape[2]
    ratio = L // _NUM_PARTS
    xb = x_ref[0]

    parts = [
        jnp.sum(xb[i * ratio:(i + 1) * ratio, :], axis=0, keepdims=True)
        for i in range(_NUM_PARTS)
    ]
    pf = jnp.concatenate(parts, axis=0) * (1.0 / ratio)

    diff = pf[:, None, :] - pf[None, :, :]
    d2 = jnp.sum(diff * diff, axis=-1)

    a = d2[:, :, None]
    bj = d2[:, None, :]
    jidx = jax.lax.broadcasted_iota(jnp.int32, (1, _NUM_PARTS, _NUM_PARTS), 2)
    midx = jax.lax.broadcasted_iota(jnp.int32, (1, _NUM_PARTS, _NUM_PARTS), 1)
    beats = (bj < a) | ((bj == a) & (jidx < midx))
    rank = jnp.sum(beats.astype(jnp.float32), axis=-1)
    adj = (rank <= 2.5).astype(jnp.float32)

    c = jnp.sum(adj, axis=0, keepdims=True) * (1.0 / ((3.0 + 1e-6) * _NUM_PARTS))
    g = jax.lax.dot_general(
        c, pf, (((1,), (0,)), ((), ())),
        preferred_element_type=jnp.float32)
    mean_pf = jnp.sum(pf, axis=0, keepdims=True) * (1.0 / _NUM_PARTS)
    out = jax.lax.dot_general(
        g, w_ref[...], (((1,), (1,)), ((), ())),
        preferred_element_type=jnp.float32)
    o_ref[0] = out + b_ref[...] + mean_pf


@jax.jit
def kernel(x, W, b):
    B, L, C = x.shape
    grid = (B,)
    out = pl.pallas_call(
        _tgam_kernel,
        grid=grid,
        in_specs=[
            pl.BlockSpec((1, L, C), lambda i: (i, 0, 0)),
            pl.BlockSpec((C, C), lambda i: (0, 0)),
            pl.BlockSpec((1, C), lambda i: (0, 0)),
        ],
        out_specs=pl.BlockSpec((1, 1, C), lambda i: (i, 0, 0)),
        out_shape=jax.ShapeDtypeStruct((B, 1, C), x.dtype),
        compiler_params=pltpu.CompilerParams(
            dimension_semantics=("arbitrary",),
        ),
    )(x, W, b.reshape(1, C))
    return out.reshape(B, C)


# speedup vs baseline: 1.5393x; 1.5393x over previous
import jax
import jax.numpy as jnp
from jax.experimental import pallas as pl
from jax.experimental.pallas import tpu as pltpu

_N = 6


def _partsum_kernel(x_ref, o_ref):
    BB, L, C = x_ref.shape
    ratio = L // _N
    for bb in range(BB):
        xb = x_ref[bb]
        parts = [
            jnp.sum(xb[i * ratio:(i + 1) * ratio, :], axis=0, keepdims=True)
            for i in range(_N)
        ]
        o_ref[bb] = jnp.concatenate(parts, axis=0)


def _finish_kernel(ps_ref, w_ref, b_ref, o_ref, *, ratio):
    B = ps_ref.shape[0]
    C = ps_ref.shape[2]
    pf = ps_ref[...] * (1.0 / ratio)

    diff = pf[:, :, None, :] - pf[:, None, :, :]
    d2 = jnp.sum(diff * diff, axis=-1)

    a = d2[:, :, :, None]
    bj = d2[:, :, None, :]
    jidx = jax.lax.broadcasted_iota(jnp.int32, (1, 1, _N, _N), 3)
    midx = jax.lax.broadcasted_iota(jnp.int32, (1, 1, _N, _N), 2)
    beats = (bj < a) | ((bj == a) & (jidx < midx))
    rank = jnp.sum(beats.astype(jnp.float32), axis=-1)
    adj = (rank <= 2.5).astype(jnp.float32)

    c = jnp.sum(adj, axis=1) * (1.0 / ((3.0 + 1e-6) * _N))
    g = jnp.sum(c[:, :, None] * pf, axis=1)
    mean_pf = jnp.sum(pf, axis=1) * (1.0 / _N)
    out = jax.lax.dot_general(
        g, w_ref[...], (((1,), (1,)), ((), ())),
        preferred_element_type=jnp.float32)
    o_ref[...] = out + b_ref[...] + mean_pf


@jax.jit
def kernel(x, W, b):
    B, L, C = x.shape
    BB = 2
    ps = pl.pallas_call(
        _partsum_kernel,
        grid=(B // BB,),
        in_specs=[pl.BlockSpec((BB, L, C), lambda i: (i, 0, 0))],
        out_specs=pl.BlockSpec((BB, _N, C), lambda i: (i, 0, 0)),
        out_shape=jax.ShapeDtypeStruct((B, _N, C), x.dtype),
        compiler_params=pltpu.CompilerParams(
            dimension_semantics=("arbitrary",),
        ),
    )(x)

    import functools
    out = pl.pallas_call(
        functools.partial(_finish_kernel, ratio=L // _N),
        in_specs=[
            pl.BlockSpec((B, _N, C), lambda: (0, 0, 0)),
            pl.BlockSpec((C, C), lambda: (0, 0)),
            pl.BlockSpec((1, C), lambda: (0, 0)),
        ],
        out_specs=pl.BlockSpec((B, C), lambda: (0, 0)),
        out_shape=jax.ShapeDtypeStruct((B, C), x.dtype),
    )(ps, W, b.reshape(1, C))
    return out


# speedup vs baseline: 1.7039x; 1.1070x over previous
import jax
import jax.numpy as jnp
from jax.experimental import pallas as pl
from jax.experimental.pallas import tpu as pltpu

_N = 6


def _partsum_kernel(x_ref, o_ref):
    BB, L, C = x_ref.shape
    ratio = L // _N
    for bb in range(BB):
        xb = x_ref[bb]
        parts = [
            jnp.sum(xb[i * ratio:(i + 1) * ratio, :], axis=0, keepdims=True)
            for i in range(_N)
        ]
        o_ref[bb] = jnp.concatenate(parts, axis=0)


def _finish_kernel(ps_ref, w_ref, b_ref, o_ref, *, ratio):
    B = ps_ref.shape[0]
    C = ps_ref.shape[2]
    pf = ps_ref[...] * (1.0 / ratio)

    diff = pf[:, :, None, :] - pf[:, None, :, :]
    d2 = jnp.sum(diff * diff, axis=-1)

    a = d2[:, :, :, None]
    bj = d2[:, :, None, :]
    jidx = jax.lax.broadcasted_iota(jnp.int32, (1, 1, _N, _N), 3)
    midx = jax.lax.broadcasted_iota(jnp.int32, (1, 1, _N, _N), 2)
    beats = (bj < a) | ((bj == a) & (jidx < midx))
    rank = jnp.sum(beats.astype(jnp.float32), axis=-1)
    adj = (rank <= 2.5).astype(jnp.float32)

    c = jnp.sum(adj, axis=1) * (1.0 / ((3.0 + 1e-6) * _N))
    g = jnp.sum(c[:, :, None] * pf, axis=1)
    mean_pf = jnp.sum(pf, axis=1) * (1.0 / _N)
    out = jax.lax.dot_general(
        g, w_ref[...], (((1,), (1,)), ((), ())),
        preferred_element_type=jnp.float32)
    o_ref[...] = out + b_ref[...] + mean_pf


@jax.jit
def kernel(x, W, b):
    B, L, C = x.shape
    BB = 4
    ps = pl.pallas_call(
        _partsum_kernel,
        grid=(B // BB,),
        in_specs=[pl.BlockSpec((BB, L, C), lambda i: (i, 0, 0))],
        out_specs=pl.BlockSpec((BB, _N, C), lambda i: (i, 0, 0)),
        out_shape=jax.ShapeDtypeStruct((B, _N, C), x.dtype),
        compiler_params=pltpu.CompilerParams(
            dimension_semantics=("arbitrary",),
        ),
    )(x)

    import functools
    out = pl.pallas_call(
        functools.partial(_finish_kernel, ratio=L // _N),
        in_specs=[
            pl.BlockSpec((B, _N, C), lambda: (0, 0, 0)),
            pl.BlockSpec((C, C), lambda: (0, 0)),
            pl.BlockSpec((1, C), lambda: (0, 0)),
        ],
        out_specs=pl.BlockSpec((B, C), lambda: (0, 0)),
        out_shape=jax.ShapeDtypeStruct((B, C), x.dtype),
    )(ps, W, b.reshape(1, C))
    return out
